# baseline (speedup 1.0000x reference)
import functools

import jax
import jax.numpy as jnp
from jax import lax
from jax.experimental import pallas as pl
from jax.experimental.pallas import tpu as pltpu
from jax.experimental.pallas import tpu_sc as plsc

B, S, D, NS = 4, 2048, 1024, 128
MAXW = 64
FSL = 128
NCHUNK = FSL // 16
WPB = D // FSL

_SIZES = [MAXW - (1 << k) + 1 for k in range(7)]
_OFFS = [sum(_SIZES[:k]) for k in range(7)]
_TROWS = sum(_SIZES)


def _pool_body(sent_hbm, esi_hbm, out_hbm,
               sev_ref, tv_ref, pv_ref, rmax_ref, rmean_ref, sem):
    nc = 2
    wid = lax.axis_index("s") * nc + lax.axis_index("c")
    b = wid // WPB
    fs = (wid % WPB) * FSL

    pltpu.sync_copy(esi_hbm.at[b], sev_ref.at[pl.ds(0, 2 * NS)])
    pltpu.sync_copy(sent_hbm.at[b, pl.ds(0, MAXW), pl.ds(fs, FSL)],
                    tv_ref.at[pl.ds(0, MAXW), :])

    zero = jnp.zeros((16,), jnp.float32)
    for c in range(NCHUNK):
        pv_ref[0, pl.ds(c * 16, 16)] = zero

    @plsc.parallel_loop(0, MAXW, unroll=2,
                        carry=tuple(zero for _ in range(NCHUNK)))
    def _(r, accs):
        new = []
        for c in range(NCHUNK):
            a = accs[c] + tv_ref[r, pl.ds(c * 16, 16)]
            pv_ref[r + 1, pl.ds(c * 16, 16)] = a
            new.append(a)
        return tuple(new)

    for k in range(1, 7):
        prev_off, off, d = _OFFS[k - 1], _OFFS[k], 1 << (k - 1)

        @plsc.parallel_loop(0, _SIZES[k], unroll=2)
        def _(i, prev_off=prev_off, off=off, d=d):
            for c in range(NCHUNK):
                lo = tv_ref[i + prev_off, pl.ds(c * 16, 16)]
                hi = tv_ref[i + prev_off + d, pl.ds(c * 16, 16)]
                tv_ref[i + off, pl.ds(c * 16, 16)] = jnp.maximum(lo, hi)

    def _query(i):
        se = sev_ref[pl.ds(2 * i, 16)]
        s_i = se[0]
        e_i = se[1]
        w = e_i - s_i + 1
        base = jnp.int32(0)
        pw = jnp.int32(1)
        for k in range(1, 7):
            cond = w >= (1 << k)
            base = jnp.where(cond, jnp.int32(_OFFS[k]), base)
            pw = jnp.where(cond, jnp.int32(1 << k), pw)
        r1 = s_i + base
        r2 = e_i + 1 - pw + base
        rv = 1.0 / jnp.full((16,), w.astype(jnp.float32))
        for c in range(NCHUNK):
            m = jnp.maximum(tv_ref[r1, pl.ds(c * 16, 16)],
                            tv_ref[r2, pl.ds(c * 16, 16)])
            rmax_ref[i, pl.ds(c * 16, 16)] = m
            sm = (pv_ref[e_i + 1, pl.ds(c * 16, 16)]
                  - pv_ref[s_i, pl.ds(c * 16, 16)])
            rmean_ref[i, pl.ds(c * 16, 16)] = sm * rv

    half = NS // 2
    plsc.parallel_loop(0, half, unroll=2)(_query)
    c1 = pltpu.async_copy(rmax_ref.at[pl.ds(0, half), :],
                          out_hbm.at[b, pl.ds(0, half), pl.ds(fs, FSL)], sem)
    c2 = pltpu.async_copy(rmean_ref.at[pl.ds(0, half), :],
                          out_hbm.at[b, pl.ds(0, half), pl.ds(D + fs, FSL)],
                          sem)
    plsc.parallel_loop(half, NS, unroll=2)(_query)
    c3 = pltpu.async_copy(rmax_ref.at[pl.ds(half, half), :],
                          out_hbm.at[b, pl.ds(half, half), pl.ds(fs, FSL)],
                          sem)
    c4 = pltpu.async_copy(rmean_ref.at[pl.ds(half, half), :],
                          out_hbm.at[b, pl.ds(half, half),
                                     pl.ds(D + fs, FSL)], sem)
    c1.wait()
    c2.wait()
    c3.wait()
    c4.wait()


def _matmul_body(cat_ref, w_ref, b_ref, o_ref):
    o_ref[...] = lax.dot_general(
        cat_ref[...].astype(jnp.bfloat16), w_ref[...].astype(jnp.bfloat16),
        dimension_numbers=(((1,), (1,)), ((), ())),
        preferred_element_type=jnp.float32,
    ) + b_ref[...]


def kernel(sentence_repr, entity_span_indices, W, b):
    esi = entity_span_indices.astype(jnp.int32).reshape(B, 2 * NS)

    pool = functools.partial(
        pl.kernel,
        mesh=plsc.VectorSubcoreMesh(core_axis_name="c", subcore_axis_name="s"),
        out_type=jax.ShapeDtypeStruct((B, NS, 2 * D), jnp.float32),
        scratch_types=[
            pltpu.VMEM((2 * NS + 16,), jnp.int32),
            pltpu.VMEM((_TROWS, FSL), jnp.float32),
            pltpu.VMEM((MAXW + 1, FSL), jnp.float32),
            pltpu.VMEM((NS, FSL), jnp.float32),
            pltpu.VMEM((NS, FSL), jnp.float32),
            pltpu.SemaphoreType.DMA,
        ],
    )(_pool_body)
    cat = pool(sentence_repr, esi)

    out = pl.pallas_call(
        _matmul_body,
        out_shape=jax.ShapeDtypeStruct((B * NS, D), jnp.float32),
    )(cat.reshape(B * NS, 2 * D), W, b.reshape(1, D))
    return out.reshape(B, NS, D)
